# baseline (speedup 1.0000x reference)
import functools

import jax
import jax.numpy as jnp
from jax import lax
from jax.experimental import pallas as pl
from jax.experimental.pallas import tpu as pltpu
from jax.experimental.pallas import tpu_sc as plsc

B, L, V, D = 1024, 200, 100000, 300

NCH = 19
PD = NCH * 16
CA, CB = 104, 96


def _sc_pool_diff(idx1_flat, idx2_flat, emb):
    info = plsc.get_sparse_core_info()
    nc, ns = info.num_cores, info.num_subcores
    nw = nc * ns
    bpw = B // nw

    mesh = plsc.VectorSubcoreMesh(core_axis_name="c", subcore_axis_name="s")

    @functools.partial(
        pl.kernel,
        out_type=jax.ShapeDtypeStruct((B * PD,), jnp.float32),
        mesh=mesh,
        scratch_types=[
            pltpu.VMEM((CA,), jnp.int32),
            pltpu.VMEM((CB,), jnp.int32),
            pltpu.VMEM((CA, PD), jnp.float32),
            pltpu.VMEM((CB, PD), jnp.float32),
            pltpu.VMEM((PD,), jnp.float32),
            pltpu.SemaphoreType.DMA,
            pltpu.SemaphoreType.DMA,
        ],
        compiler_params=pltpu.CompilerParams(use_tc_tiling_on_sc=False),
    )
    def sc_kernel(i1_hbm, i2_hbm, emb_hbm, out_hbm,
                  idxa, idxb, bufa, bufb, acc, sema, semb):
        wid = lax.axis_index("s") * nc + lax.axis_index("c")
        base_b = wid * bpw

        def accum_rows(buf, nrows, sign):
            def row_body(r, _):
                for c in range(NCH):
                    co = c * 16
                    v = buf[r, pl.ds(co, 16)]
                    if sign < 0:
                        v = -v
                    plsc.addupdate(acc.at[pl.ds(c * 16, 16)], v)
                return _
            lax.fori_loop(0, nrows, row_body, None)

        def batch_body(bl, _):
            b = base_b + bl
            zero = jnp.zeros((16,), jnp.float32)
            for c in range(NCH):
                acc[pl.ds(c * 16, 16)] = zero

            for i_hbm, sign in ((i1_hbm, 1), (i2_hbm, -1)):
                pltpu.sync_copy(i_hbm.at[pl.ds(b * L, CA)], idxa)
                pltpu.sync_copy(i_hbm.at[pl.ds(b * L + CA, CB)], idxb)
                cpa = pltpu.async_copy(emb_hbm.at[idxa], bufa, sema)
                cpb = pltpu.async_copy(emb_hbm.at[idxb], bufb, semb)
                cpa.wait()
                accum_rows(bufa, CA, sign)
                cpb.wait()
                accum_rows(bufb, CB, sign)

            pltpu.sync_copy(acc, out_hbm.at[pl.ds(b * PD, PD)])
            return _

        lax.fori_loop(0, bpw, batch_body, None)

    return sc_kernel(idx1_flat, idx2_flat, emb)


def _mlp(pooled, w1e, b1, w2, b2, w3, b3, w4, b4, w5, b5):
    def body(p_ref, w1_ref, b1_ref, w2_ref, b2_ref, w3_ref, b3_ref,
             w4_ref, b4_ref, w5_ref, b5_ref, out_ref):
        def leaky(x):
            return jnp.where(x >= 0, x, 10.0 * x)
        x = p_ref[...]
        x = leaky(jnp.dot(x, w1_ref[...], preferred_element_type=jnp.float32)
                  + b1_ref[...])
        x = leaky(jnp.dot(x, w2_ref[...], preferred_element_type=jnp.float32)
                  + b2_ref[...])
        x = leaky(jnp.dot(x, w3_ref[...], preferred_element_type=jnp.float32)
                  + b3_ref[...])
        x = leaky(jnp.dot(x, w4_ref[...], preferred_element_type=jnp.float32)
                  + b4_ref[...])
        x = jnp.dot(x, w5_ref[...], preferred_element_type=jnp.float32) \
            + b5_ref[...]
        out_ref[...] = x

    return pl.pallas_call(
        body,
        out_shape=jax.ShapeDtypeStruct((B, 2), jnp.float32),
    )(pooled, w1e, b1, w2, b2, w3, b3, w4, b4, w5, b5)


def kernel(input_1, input_2, emb, W1, b1, W2, b2, W3, b3, W4, b4, W5, b5):
    i1 = input_1.reshape(-1).astype(jnp.int32)
    i2 = input_2.reshape(-1).astype(jnp.int32)

    embp = jnp.pad(emb, ((0, 0), (0, PD - D)))
    pooled = _sc_pool_diff(i1, i2, embp).reshape(B, PD)

    W1b = W1[D:2 * D] * (1.0 / L)
    w1e = jnp.concatenate([W1b, jnp.zeros((PD - D, 2 * D), W1.dtype)], axis=0)

    return _mlp(pooled, w1e, b1, W2, b2, W3, b3, W4, b4, W5, b5)


# speedup vs baseline: 2.3008x; 2.3008x over previous
import functools

import jax
import jax.numpy as jnp
from jax import lax
from jax.experimental import pallas as pl
from jax.experimental.pallas import tpu as pltpu
from jax.experimental.pallas import tpu_sc as plsc

B, L, V, D = 1024, 200, 100000, 300

NCH = 19
PD = NCH * 16
DP = 384
CA, CB = 104, 96


def _sc_pool_diff(idx1_flat, idx2_flat, embp):
    info = plsc.get_sparse_core_info()
    nc, ns = info.num_cores, info.num_subcores
    nw = nc * ns
    bpw = B // nw

    mesh = plsc.VectorSubcoreMesh(core_axis_name="c", subcore_axis_name="s")

    @functools.partial(
        pl.kernel,
        out_type=jax.ShapeDtypeStruct((B * PD,), jnp.float32),
        mesh=mesh,
        scratch_types=[
            pltpu.VMEM((CA,), jnp.int32),
            pltpu.VMEM((CB,), jnp.int32),
            pltpu.VMEM((CA, DP), jnp.float32),
            pltpu.VMEM((CB, DP), jnp.float32),
            pltpu.VMEM((PD,), jnp.float32),
            pltpu.SemaphoreType.DMA,
            pltpu.SemaphoreType.DMA,
        ],
    )
    def sc_kernel(i1_hbm, i2_hbm, emb_hbm, out_hbm,
                  idx0, idx1, buf0, buf1, stg, sem0, sem1):
        wid = lax.axis_index("s") * nc + lax.axis_index("c")
        base_b = wid * bpw

        idxs = (idx0, idx1)
        bufs = (buf0, buf1)
        sems = (sem0, sem1)
        items = ((0, 0, CA), (0, CA, CB), (1, 0, CA), (1, CA, CB))

        def stage_and_fire(k, b):
            src, off, rows = items[k]
            i_hbm = i1_hbm if src == 0 else i2_hbm
            sl = k % 2
            pltpu.sync_copy(i_hbm.at[pl.ds(b * L + off, rows)], idxs[sl])
            pltpu.async_copy(emb_hbm.at[idxs[sl]], bufs[sl], sems[sl])

        def accum(k, acc):
            _, _, rows = items[k]
            buf = bufs[k % 2]
            sign = items[k][0] == 0

            def row_body(r, acc):
                if sign:
                    return tuple(
                        acc[c] + buf[r, pl.ds(c * 16, 16)]
                        for c in range(NCH))
                return tuple(
                    acc[c] - buf[r, pl.ds(c * 16, 16)]
                    for c in range(NCH))
            return lax.fori_loop(0, rows, row_body, acc)

        def batch_body(bl, _):
            b = base_b + bl
            acc = tuple(jnp.zeros((16,), jnp.float32) for _ in range(NCH))
            for k in range(4):
                if k < 3:
                    stage_and_fire(k + 1, b)
                else:
                    @pl.when(bl < bpw - 1)
                    def _():
                        stage_and_fire(0, b + 1)
                sl = k % 2
                pltpu.make_async_copy(
                    emb_hbm.at[idxs[sl]], bufs[sl], sems[sl]).wait()
                acc = accum(k, acc)
            for c in range(NCH):
                stg[pl.ds(c * 16, 16)] = acc[c]
            pltpu.sync_copy(stg, out_hbm.at[pl.ds(b * PD, PD)])
            return _

        stage_and_fire(0, base_b)
        lax.fori_loop(0, bpw, batch_body, None)

    return sc_kernel(idx1_flat, idx2_flat, embp)


def _mlp(pooled, w1e, b1, w2, b2, w3, b3, w4, b4, w5, b5):
    def body(p_ref, w1_ref, b1_ref, w2_ref, b2_ref, w3_ref, b3_ref,
             w4_ref, b4_ref, w5_ref, b5_ref, out_ref):
        def leaky(x):
            return jnp.where(x >= 0, x, 10.0 * x)
        x = p_ref[...]
        x = leaky(jnp.dot(x, w1_ref[...], preferred_element_type=jnp.float32)
                  + b1_ref[...])
        x = leaky(jnp.dot(x, w2_ref[...], preferred_element_type=jnp.float32)
                  + b2_ref[...])
        x = leaky(jnp.dot(x, w3_ref[...], preferred_element_type=jnp.float32)
                  + b3_ref[...])
        x = leaky(jnp.dot(x, w4_ref[...], preferred_element_type=jnp.float32)
                  + b4_ref[...])
        x = jnp.dot(x, w5_ref[...], preferred_element_type=jnp.float32) \
            + b5_ref[...]
        out_ref[...] = x

    return pl.pallas_call(
        body,
        out_shape=jax.ShapeDtypeStruct((B, 2), jnp.float32),
    )(pooled, w1e, b1, w2, b2, w3, b3, w4, b4, w5, b5)


def kernel(input_1, input_2, emb, W1, b1, W2, b2, W3, b3, W4, b4, W5, b5):
    i1 = input_1.reshape(-1).astype(jnp.int32)
    i2 = input_2.reshape(-1).astype(jnp.int32)

    embp = jnp.pad(emb, ((0, 0), (0, DP - D)))
    pooled = _sc_pool_diff(i1, i2, embp).reshape(B, PD)

    W1b = W1[D:2 * D] * (1.0 / L)
    w1e = jnp.concatenate([W1b, jnp.zeros((PD - D, 2 * D), W1.dtype)], axis=0)

    return _mlp(pooled, w1e, b1, W2, b2, W3, b3, W4, b4, W5, b5)


# speedup vs baseline: 3.7704x; 1.6387x over previous
import functools

import jax
import jax.numpy as jnp
from jax import lax
from jax.experimental import pallas as pl
from jax.experimental.pallas import tpu as pltpu
from jax.experimental.pallas import tpu_sc as plsc

B, L, V, D = 1024, 200, 100000, 300

NCH = 19
PD = NCH * 16
DP = 384
CA, CB = 104, 96


def _sc_pool_diff(idx1_flat, idx2_flat, embp):
    info = plsc.get_sparse_core_info()
    nc, ns = info.num_cores, info.num_subcores
    nw = nc * ns
    bpw = B // nw

    mesh = plsc.VectorSubcoreMesh(core_axis_name="c", subcore_axis_name="s")

    @functools.partial(
        pl.kernel,
        out_type=jax.ShapeDtypeStruct((B * PD,), jnp.float32),
        mesh=mesh,
        scratch_types=[
            pltpu.VMEM((CA,), jnp.int32),
            pltpu.VMEM((CB,), jnp.int32),
            pltpu.VMEM((CA, DP), jnp.float32),
            pltpu.VMEM((CB, DP), jnp.float32),
            pltpu.VMEM((PD,), jnp.float32),
            pltpu.SemaphoreType.DMA,
            pltpu.SemaphoreType.DMA,
        ],
    )
    def sc_kernel(i1_hbm, i2_hbm, emb_hbm, out_hbm,
                  idx0, idx1, buf0, buf1, stg, sem0, sem1):
        wid = lax.axis_index("s") * nc + lax.axis_index("c")
        base_b = wid * bpw

        idxs = (idx0, idx1)
        bufs = (buf0, buf1)
        sems = (sem0, sem1)
        items = ((0, 0, CA), (0, CA, CB), (1, 0, CA), (1, CA, CB))

        def stage_and_fire(k, b):
            src, off, rows = items[k]
            i_hbm = i1_hbm if src == 0 else i2_hbm
            sl = k % 2
            pltpu.sync_copy(i_hbm.at[pl.ds(b * L + off, rows)], idxs[sl])
            pltpu.async_copy(emb_hbm.at[idxs[sl]], bufs[sl], sems[sl])

        def accum(k, acc):
            _, _, rows = items[k]
            buf = bufs[k % 2]
            sign = items[k][0] == 0

            def row_body(r, acc):
                if sign:
                    return tuple(
                        acc[c] + buf[r, pl.ds(c * 16, 16)]
                        for c in range(NCH))
                return tuple(
                    acc[c] - buf[r, pl.ds(c * 16, 16)]
                    for c in range(NCH))
            return lax.fori_loop(0, rows, row_body, acc)

        def batch_body(bl, _):
            b = base_b + bl
            acc = tuple(jnp.zeros((16,), jnp.float32) for _ in range(NCH))
            for k in range(4):
                if k < 3:
                    stage_and_fire(k + 1, b)
                else:
                    @pl.when(bl < bpw - 1)
                    def _():
                        stage_and_fire(0, b + 1)
                sl = k % 2
                pltpu.make_async_copy(
                    emb_hbm.at[idxs[sl]], bufs[sl], sems[sl]).wait()
                acc = accum(k, acc)
            for c in range(NCH):
                stg[pl.ds(c * 16, 16)] = acc[c]
            pltpu.sync_copy(stg, out_hbm.at[pl.ds(b * PD, PD)])
            return _

        stage_and_fire(0, base_b)
        lax.fori_loop(0, bpw, batch_body, None)

    return sc_kernel(idx1_flat, idx2_flat, embp)


def _pad_table(emb):
    nblk = 100
    rb = V // nblk

    def body(in_ref, out_ref):
        out_ref[...] = jnp.pad(in_ref[...], ((0, 0), (0, DP - D)))

    return pl.pallas_call(
        body,
        grid=(nblk,),
        in_specs=[pl.BlockSpec((rb, D), lambda i: (i, 0))],
        out_specs=pl.BlockSpec((rb, DP), lambda i: (i, 0)),
        out_shape=jax.ShapeDtypeStruct((V, DP), jnp.float32),
    )(emb)


def _mlp(pooled, w1e, b1, w2, b2, w3, b3, w4, b4, w5, b5):
    def body(p_ref, w1_ref, b1_ref, w2_ref, b2_ref, w3_ref, b3_ref,
             w4_ref, b4_ref, w5_ref, b5_ref, out_ref):
        def leaky(x):
            return jnp.where(x >= 0, x, 10.0 * x)
        x = p_ref[...]
        x = leaky(jnp.dot(x, w1_ref[...], preferred_element_type=jnp.float32)
                  + b1_ref[...])
        x = leaky(jnp.dot(x, w2_ref[...], preferred_element_type=jnp.float32)
                  + b2_ref[...])
        x = leaky(jnp.dot(x, w3_ref[...], preferred_element_type=jnp.float32)
                  + b3_ref[...])
        x = leaky(jnp.dot(x, w4_ref[...], preferred_element_type=jnp.float32)
                  + b4_ref[...])
        x = jnp.dot(x, w5_ref[...], preferred_element_type=jnp.float32) \
            + b5_ref[...]
        out_ref[...] = x

    return pl.pallas_call(
        body,
        out_shape=jax.ShapeDtypeStruct((B, 2), jnp.float32),
    )(pooled, w1e, b1, w2, b2, w3, b3, w4, b4, w5, b5)


def kernel(input_1, input_2, emb, W1, b1, W2, b2, W3, b3, W4, b4, W5, b5):
    i1 = input_1.reshape(-1).astype(jnp.int32)
    i2 = input_2.reshape(-1).astype(jnp.int32)

    embp = _pad_table(emb)
    pooled = _sc_pool_diff(i1, i2, embp).reshape(B, PD)

    W1b = W1[D:2 * D] * (1.0 / L)
    w1e = jnp.concatenate([W1b, jnp.zeros((PD - D, 2 * D), W1.dtype)], axis=0)

    return _mlp(pooled, w1e, b1, W2, b2, W3, b3, W4, b4, W5, b5)
